# baseline (speedup 1.0000x reference)
import functools

import jax
import jax.numpy as jnp
from jax import lax
from jax.experimental import pallas as pl
from jax.experimental.pallas import tpu as pltpu
from jax.experimental.pallas import tpu_sc as plsc

_NC = 2
_NS = 16
_NW = _NC * _NS
_LANES = 16


def _make_hist_sqdiff(n_pad: int, chunk: int, h: int, w: int, c: int, hsc: int):
    hw = h * w
    per_tile = hw // _NW
    rpt = hsc // _NW
    row0_base = h - hsc
    assert hw % _NW == 0 and chunk % _LANES == 0 and n_pad % chunk == 0
    assert hsc % _NW == 0 and w % _LANES == 0

    @functools.partial(
        pl.kernel,
        mesh=plsc.VectorSubcoreMesh(core_axis_name="c", subcore_axis_name="s"),
        compiler_params=pltpu.CompilerParams(needs_layout_passes=False),
        out_type=(
            jax.ShapeDtypeStruct((hw,), jnp.float32),
            jax.ShapeDtypeStruct((hsc, w), jnp.float32),
        ),
        scratch_types=[
            pltpu.VMEM((chunk * 2,), jnp.int32),
            pltpu.VMEM((per_tile,), jnp.float32),
            pltpu.VMEM((rpt, w), jnp.float32),
            pltpu.VMEM((rpt, w), jnp.float32),
            pltpu.VMEM((rpt, w), jnp.float32),
            pltpu.VMEM((rpt, w), jnp.float32),
            pltpu.VMEM((rpt, w), jnp.float32),
            pltpu.SemaphoreType.DMA,
            pltpu.SemaphoreType.DMA,
        ],
    )
    def hist_sq(dots_hbm, img_hbm, rew_hbm, cnt_hbm, sbot_hbm,
                dots_v, hist_v, ib0, rb0, ib1, rb1, acc_v, sem0, sem1):
        cid = lax.axis_index("c")
        sid = lax.axis_index("s")
        wid = sid * _NC + cid
        base = wid * per_tile
        row0 = row0_base + wid * rpt

        def fire(ch, ib, rb, sem):
            pltpu.async_copy(img_hbm.at[ch, pl.ds(row0, rpt)], ib, sem)
            pltpu.async_copy(rew_hbm.at[ch, pl.ds(row0, rpt)], rb, sem)

        def drain(ib, rb, sem):
            pltpu.make_async_copy(img_hbm.at[0, pl.ds(row0, rpt)], ib, sem).wait()
            pltpu.make_async_copy(rew_hbm.at[0, pl.ds(row0, rpt)], rb, sem).wait()

        fire(0, ib0, rb0, sem0)
        fire(1, ib1, rb1, sem1)

        zeros16 = jnp.zeros((_LANES,), jnp.float32)

        def zbody(k, carry):
            hist_v[pl.ds(k * _LANES, _LANES)] = zeros16
            return carry

        lax.fori_loop(0, per_tile // _LANES, zbody, 0)

        iota2 = lax.iota(jnp.int32, _LANES) * 2
        ones16 = jnp.ones((_LANES,), jnp.float32)

        for ci in range(n_pad // chunk):
            pltpu.sync_copy(dots_hbm.at[pl.ds(ci * chunk * 2, chunk * 2)], dots_v)

            def body(i, carry):
                off = i * (2 * _LANES) + iota2
                xs = plsc.load_gather(dots_v, [off])
                ys = plsc.load_gather(dots_v, [off + 1])
                local = ys * w + xs - base
                mask = (local >= 0) & (local < per_tile)
                safe = jnp.minimum(jnp.maximum(local, 0), per_tile - 1)
                plsc.addupdate_scatter(hist_v, [safe], ones16, mask=mask)
                return carry

            lax.fori_loop(0, chunk // _LANES, body, 0)

        pltpu.sync_copy(hist_v, cnt_hbm.at[pl.ds(base, per_tile)])

        for r in range(rpt):
            for g in range(w // _LANES):
                acc_v[r, pl.ds(g * _LANES, _LANES)] = zeros16

        def accumulate(ib, rb):
            for r in range(rpt):
                for g in range(w // _LANES):
                    sl = pl.ds(g * _LANES, _LANES)
                    d = ib[r, sl] - rb[r, sl]
                    plsc.addupdate(acc_v.at[r, sl], d * d)

        def cbody(i, carry):
            drain(ib0, rb0, sem0)
            accumulate(ib0, rb0)
            fire(2 * i + 2, ib0, rb0, sem0)
            drain(ib1, rb1, sem1)
            accumulate(ib1, rb1)
            fire(2 * i + 3, ib1, rb1, sem1)
            return carry

        lax.fori_loop(0, (c - 2) // 2, cbody, 0)
        drain(ib0, rb0, sem0)
        accumulate(ib0, rb0)
        drain(ib1, rb1, sem1)
        accumulate(ib1, rb1)

        pltpu.sync_copy(acc_v, sbot_hbm.at[pl.ds(wid * rpt, rpt)])

    return hist_sq


def _sqdiff_body(img_ref, rew_ref, s_ref):
    d = img_ref[...] - rew_ref[...]
    s_ref[...] = jnp.sum(d * d, axis=0)


def _wsum_body(stop_ref, sbot_ref, cnt_ref, tot_ref):
    htop = stop_ref.shape[0]
    t = jnp.sum(stop_ref[...] * cnt_ref[:htop, :])
    b = jnp.sum(sbot_ref[...] * cnt_ref[htop:, :])
    tot_ref[0, 0] = t + b


def kernel(image, image_rewrite, dot_list_format):
    c, h, w = image.shape
    n = dot_list_format.shape[0]
    hw = h * w

    chunk = 10000
    if chunk % _LANES:
        chunk = ((chunk // _LANES) + 1) * _LANES
    n_pad = ((n + chunk - 1) // chunk) * chunk
    dots = dot_list_format
    if n_pad != n:
        fill = jnp.concatenate(
            [
                jnp.zeros((n_pad - n, 1), jnp.int32),
                jnp.full((n_pad - n, 1), h, jnp.int32),
            ],
            axis=1,
        )
        dots = jnp.concatenate([dots, fill], axis=0)
    dots_flat = dots.reshape(n_pad * 2)

    hsc = _NW * max(1, (h // 4) // _NW)
    counts, s_bot = _make_hist_sqdiff(n_pad, chunk, h, w, c, hsc)(
        dots_flat, image, image_rewrite
    )
    counts2d = counts.reshape(h, w)

    htop = h - hsc
    br = 64
    s_top = pl.pallas_call(
        _sqdiff_body,
        grid=(htop // br,),
        in_specs=[
            pl.BlockSpec((c, br, w), lambda i: (0, i, 0)),
            pl.BlockSpec((c, br, w), lambda i: (0, i, 0)),
        ],
        out_specs=pl.BlockSpec((br, w), lambda i: (i, 0)),
        out_shape=jax.ShapeDtypeStruct((htop, w), jnp.float32),
    )(image, image_rewrite)

    tot = pl.pallas_call(
        _wsum_body,
        in_specs=[
            pl.BlockSpec((htop, w), lambda: (0, 0)),
            pl.BlockSpec((hsc, w), lambda: (0, 0)),
            pl.BlockSpec((h, w), lambda: (0, 0)),
        ],
        out_specs=pl.BlockSpec(memory_space=pltpu.SMEM),
        out_shape=jax.ShapeDtypeStruct((1, 1), jnp.float32),
    )(s_top, s_bot, counts2d)

    return tot[0, 0] / jnp.float32(n)


# speedup vs baseline: 1.4339x; 1.4339x over previous
import functools

import jax
import jax.numpy as jnp
from jax import lax
from jax.experimental import pallas as pl
from jax.experimental.pallas import tpu as pltpu
from jax.experimental.pallas import tpu_sc as plsc

_NC = 2
_NS = 16
_NW = _NC * _NS
_LANES = 16


def _make_hist_sqdiff(n_pad: int, chunk: int, h: int, w: int, c: int, hsc: int):
    hw = h * w
    per_tile = hw // _NW
    rpt = hsc // _NW
    row0_base = h - hsc
    assert hw % _NW == 0 and chunk % _LANES == 0 and n_pad % chunk == 0
    assert hsc % _NW == 0 and w % _LANES == 0

    @functools.partial(
        pl.kernel,
        mesh=plsc.VectorSubcoreMesh(core_axis_name="c", subcore_axis_name="s"),
        compiler_params=pltpu.CompilerParams(needs_layout_passes=False),
        out_type=(
            jax.ShapeDtypeStruct((hw,), jnp.float32),
            jax.ShapeDtypeStruct((hsc, w), jnp.float32),
        ),
        scratch_types=[
            pltpu.VMEM((chunk * 2,), jnp.int32),
            pltpu.VMEM((per_tile,), jnp.float32),
            [pltpu.VMEM((rpt, w), jnp.float32) for _ in range(4)],
            [pltpu.VMEM((rpt, w), jnp.float32) for _ in range(4)],
            pltpu.VMEM((rpt, w), jnp.float32),
            [pltpu.SemaphoreType.DMA for _ in range(4)],
        ],
    )
    def hist_sq(dots_hbm, img_hbm, rew_hbm, cnt_hbm, sbot_hbm,
                dots_v, hist_v, ibs, rbs, acc_v, sems):
        cid = lax.axis_index("c")
        sid = lax.axis_index("s")
        wid = sid * _NC + cid
        base = wid * per_tile
        row0 = row0_base + wid * rpt

        def fire(ch, k):
            pltpu.async_copy(img_hbm.at[ch, pl.ds(row0, rpt)], ibs[k], sems[k])
            pltpu.async_copy(rew_hbm.at[ch, pl.ds(row0, rpt)], rbs[k], sems[k])

        def drain(k):
            pltpu.make_async_copy(
                img_hbm.at[0, pl.ds(row0, rpt)], ibs[k], sems[k]
            ).wait()
            pltpu.make_async_copy(
                rew_hbm.at[0, pl.ds(row0, rpt)], rbs[k], sems[k]
            ).wait()

        for k in range(4):
            fire(k, k)

        zeros16 = jnp.zeros((_LANES,), jnp.float32)

        def zbody(k, carry):
            hist_v[pl.ds(k * _LANES, _LANES)] = zeros16
            return carry

        lax.fori_loop(0, per_tile // _LANES, zbody, 0)

        iota2 = lax.iota(jnp.int32, _LANES) * 2
        ones16 = jnp.ones((_LANES,), jnp.float32)

        for ci in range(n_pad // chunk):
            pltpu.sync_copy(dots_hbm.at[pl.ds(ci * chunk * 2, chunk * 2)], dots_v)

            def body(i, carry):
                off = i * (2 * _LANES) + iota2
                xs = plsc.load_gather(dots_v, [off])
                ys = plsc.load_gather(dots_v, [off + 1])
                local = ys * w + xs - base
                mask = (local >= 0) & (local < per_tile)
                safe = jnp.minimum(jnp.maximum(local, 0), per_tile - 1)
                plsc.addupdate_scatter(hist_v, [safe], ones16, mask=mask)
                return carry

            lax.fori_loop(0, chunk // _LANES, body, 0)

        pltpu.sync_copy(hist_v, cnt_hbm.at[pl.ds(base, per_tile)])

        for r in range(rpt):

            @plsc.parallel_loop(0, w // _LANES, unroll=8)
            def _(g):
                acc_v[r, pl.ds(g * _LANES, _LANES)] = zeros16

        def accumulate(ib, rb):
            for r in range(rpt):

                @plsc.parallel_loop(0, w // _LANES, unroll=8)
                def _(g):
                    sl = pl.ds(g * _LANES, _LANES)
                    d = ib[r, sl] - rb[r, sl]
                    plsc.addupdate(acc_v.at[r, sl], d * d)

        def cbody(i, carry):
            for k in range(4):
                drain(k)
                accumulate(ibs[k], rbs[k])
                fire(4 * i + 4 + k, k)
            return carry

        lax.fori_loop(0, (c - 4) // 4, cbody, 0)
        for k in range(4):
            drain(k)
            accumulate(ibs[k], rbs[k])

        pltpu.sync_copy(acc_v, sbot_hbm.at[pl.ds(wid * rpt, rpt)])

    return hist_sq


def _sqdiff_body(img_ref, rew_ref, s_ref):
    d = img_ref[...] - rew_ref[...]
    s_ref[...] = jnp.sum(d * d, axis=0)


def _wsum_body(stop_ref, sbot_ref, cnt_ref, tot_ref):
    htop = stop_ref.shape[0]
    t = jnp.sum(stop_ref[...] * cnt_ref[:htop, :])
    b = jnp.sum(sbot_ref[...] * cnt_ref[htop:, :])
    tot_ref[0, 0] = t + b


def kernel(image, image_rewrite, dot_list_format):
    c, h, w = image.shape
    n = dot_list_format.shape[0]
    hw = h * w

    chunk = 10000
    if chunk % _LANES:
        chunk = ((chunk // _LANES) + 1) * _LANES
    n_pad = ((n + chunk - 1) // chunk) * chunk
    dots = dot_list_format
    if n_pad != n:
        fill = jnp.concatenate(
            [
                jnp.zeros((n_pad - n, 1), jnp.int32),
                jnp.full((n_pad - n, 1), h, jnp.int32),
            ],
            axis=1,
        )
        dots = jnp.concatenate([dots, fill], axis=0)
    dots_flat = dots.reshape(n_pad * 2)

    hsc = _NW * max(1, (h // 4) // _NW)
    counts, s_bot = _make_hist_sqdiff(n_pad, chunk, h, w, c, hsc)(
        dots_flat, image, image_rewrite
    )
    counts2d = counts.reshape(h, w)

    htop = h - hsc
    br = 64
    s_top = pl.pallas_call(
        _sqdiff_body,
        grid=(htop // br,),
        in_specs=[
            pl.BlockSpec((c, br, w), lambda i: (0, i, 0)),
            pl.BlockSpec((c, br, w), lambda i: (0, i, 0)),
        ],
        out_specs=pl.BlockSpec((br, w), lambda i: (i, 0)),
        out_shape=jax.ShapeDtypeStruct((htop, w), jnp.float32),
    )(image, image_rewrite)

    tot = pl.pallas_call(
        _wsum_body,
        in_specs=[
            pl.BlockSpec((htop, w), lambda: (0, 0)),
            pl.BlockSpec((hsc, w), lambda: (0, 0)),
            pl.BlockSpec((h, w), lambda: (0, 0)),
        ],
        out_specs=pl.BlockSpec(memory_space=pltpu.SMEM),
        out_shape=jax.ShapeDtypeStruct((1, 1), jnp.float32),
    )(s_top, s_bot, counts2d)

    return tot[0, 0] / jnp.float32(n)


# speedup vs baseline: 2.4008x; 1.6744x over previous
import functools

import jax
import jax.numpy as jnp
from jax import lax
from jax.experimental import pallas as pl
from jax.experimental.pallas import tpu as pltpu
from jax.experimental.pallas import tpu_sc as plsc

_NC = 2
_NS = 16
_NW = _NC * _NS
_LANES = 16


def _make_hist_sqdiff(n_pad: int, chunk: int, h: int, w: int, c: int, hsc: int):
    hw = h * w
    per_tile = hw // _NW
    rpt = hsc // _NW
    row0_base = h - hsc
    assert hw % _NW == 0 and chunk % _LANES == 0 and n_pad % chunk == 0
    assert hsc % _NW == 0 and w % _LANES == 0

    @functools.partial(
        pl.kernel,
        mesh=plsc.VectorSubcoreMesh(core_axis_name="c", subcore_axis_name="s"),
        compiler_params=pltpu.CompilerParams(needs_layout_passes=False),
        out_type=(
            jax.ShapeDtypeStruct((hw,), jnp.float32),
            jax.ShapeDtypeStruct((hsc, w), jnp.float32),
        ),
        scratch_types=[
            pltpu.VMEM((chunk,), jnp.int32),
            pltpu.VMEM((chunk,), jnp.int32),
            pltpu.VMEM((per_tile,), jnp.float32),
            [pltpu.VMEM((rpt, w), jnp.float32) for _ in range(4)],
            [pltpu.VMEM((rpt, w), jnp.float32) for _ in range(4)],
            pltpu.VMEM((rpt, w), jnp.float32),
            [pltpu.SemaphoreType.DMA for _ in range(4)],
        ],
    )
    def hist_sq(xs_hbm, ys_hbm, img_hbm, rew_hbm, cnt_hbm, sbot_hbm,
                xs_v, ys_v, hist_v, ibs, rbs, acc_v, sems):
        cid = lax.axis_index("c")
        sid = lax.axis_index("s")
        wid = sid * _NC + cid
        base = wid * per_tile
        row0 = row0_base + wid * rpt

        def fire(ch, k):
            pltpu.async_copy(img_hbm.at[ch, pl.ds(row0, rpt)], ibs[k], sems[k])
            pltpu.async_copy(rew_hbm.at[ch, pl.ds(row0, rpt)], rbs[k], sems[k])

        def drain(k):
            pltpu.make_async_copy(
                img_hbm.at[0, pl.ds(row0, rpt)], ibs[k], sems[k]
            ).wait()
            pltpu.make_async_copy(
                rew_hbm.at[0, pl.ds(row0, rpt)], rbs[k], sems[k]
            ).wait()

        for k in range(4):
            fire(k, k)

        zeros16 = jnp.zeros((_LANES,), jnp.float32)

        def zbody(k, carry):
            hist_v[pl.ds(k * _LANES, _LANES)] = zeros16
            return carry

        lax.fori_loop(0, per_tile // _LANES, zbody, 0)

        ones16 = jnp.ones((_LANES,), jnp.float32)

        for ci in range(n_pad // chunk):
            pltpu.sync_copy(xs_hbm.at[pl.ds(ci * chunk, chunk)], xs_v)
            pltpu.sync_copy(ys_hbm.at[pl.ds(ci * chunk, chunk)], ys_v)

            @plsc.parallel_loop(0, chunk // _LANES, unroll=8)
            def _(i):
                sl = pl.ds(i * _LANES, _LANES)
                local = ys_v[sl] * w + xs_v[sl] - base
                mask = (local >= 0) & (local < per_tile)
                safe = jnp.minimum(jnp.maximum(local, 0), per_tile - 1)
                plsc.addupdate_scatter(hist_v, [safe], ones16, mask=mask)

        pltpu.sync_copy(hist_v, cnt_hbm.at[pl.ds(base, per_tile)])

        for r in range(rpt):

            @plsc.parallel_loop(0, w // _LANES, unroll=8)
            def _(g):
                acc_v[r, pl.ds(g * _LANES, _LANES)] = zeros16

        def accumulate(ib, rb):
            for r in range(rpt):

                @plsc.parallel_loop(0, w // _LANES, unroll=8)
                def _(g):
                    sl = pl.ds(g * _LANES, _LANES)
                    d = ib[r, sl] - rb[r, sl]
                    plsc.addupdate(acc_v.at[r, sl], d * d)

        def cbody(i, carry):
            for k in range(4):
                drain(k)
                accumulate(ibs[k], rbs[k])
                fire(4 * i + 4 + k, k)
            return carry

        lax.fori_loop(0, (c - 4) // 4, cbody, 0)
        for k in range(4):
            drain(k)
            accumulate(ibs[k], rbs[k])

        pltpu.sync_copy(acc_v, sbot_hbm.at[pl.ds(wid * rpt, rpt)])

    return hist_sq


def _sqdiff_body(img_ref, rew_ref, s_ref):
    d = img_ref[...] - rew_ref[...]
    s_ref[...] = jnp.sum(d * d, axis=0)


def _wsum_body(stop_ref, sbot_ref, cnt_ref, tot_ref):
    htop = stop_ref.shape[0]
    t = jnp.sum(stop_ref[...] * cnt_ref[:htop, :])
    b = jnp.sum(sbot_ref[...] * cnt_ref[htop:, :])
    tot_ref[0, 0] = t + b


def kernel(image, image_rewrite, dot_list_format):
    c, h, w = image.shape
    n = dot_list_format.shape[0]
    hw = h * w

    chunk = 10000
    if chunk % _LANES:
        chunk = ((chunk // _LANES) + 1) * _LANES
    n_pad = ((n + chunk - 1) // chunk) * chunk
    dots_t = dot_list_format.T
    if n_pad != n:
        fill = jnp.concatenate(
            [
                jnp.zeros((1, n_pad - n), jnp.int32),
                jnp.full((1, n_pad - n), h, jnp.int32),
            ],
            axis=0,
        )
        dots_t = jnp.concatenate([dots_t, fill], axis=1)

    hsc = _NW * max(1, (h // 4) // _NW)
    counts, s_bot = _make_hist_sqdiff(n_pad, chunk, h, w, c, hsc)(
        dots_t[0], dots_t[1], image, image_rewrite
    )
    counts2d = counts.reshape(h, w)

    htop = h - hsc
    br = 32
    s_top = pl.pallas_call(
        _sqdiff_body,
        grid=(htop // br,),
        in_specs=[
            pl.BlockSpec((c, br, w), lambda i: (0, i, 0)),
            pl.BlockSpec((c, br, w), lambda i: (0, i, 0)),
        ],
        out_specs=pl.BlockSpec((br, w), lambda i: (i, 0)),
        out_shape=jax.ShapeDtypeStruct((htop, w), jnp.float32),
    )(image, image_rewrite)

    tot = pl.pallas_call(
        _wsum_body,
        in_specs=[
            pl.BlockSpec((htop, w), lambda: (0, 0)),
            pl.BlockSpec((hsc, w), lambda: (0, 0)),
            pl.BlockSpec((h, w), lambda: (0, 0)),
        ],
        out_specs=pl.BlockSpec(memory_space=pltpu.SMEM),
        out_shape=jax.ShapeDtypeStruct((1, 1), jnp.float32),
    )(s_top, s_bot, counts2d)

    return tot[0, 0] / jnp.float32(n)


# speedup vs baseline: 2.7908x; 1.1625x over previous
import functools

import jax
import jax.numpy as jnp
from jax import lax
from jax.experimental import pallas as pl
from jax.experimental.pallas import tpu as pltpu
from jax.experimental.pallas import tpu_sc as plsc

_NC = 2
_NS = 16
_NW = _NC * _NS
_LANES = 16


def _make_hist_sqdiff(n_pad: int, chunk: int, h: int, w: int, c: int, hsc: int):
    hw = h * w
    per_tile = hw // _NW
    rpt = hsc // _NW
    row0_base = h - hsc
    assert hw % _NW == 0 and chunk % _LANES == 0 and n_pad % chunk == 0
    assert hsc % _NW == 0 and w % _LANES == 0

    @functools.partial(
        pl.kernel,
        mesh=plsc.VectorSubcoreMesh(core_axis_name="c", subcore_axis_name="s"),
        compiler_params=pltpu.CompilerParams(needs_layout_passes=False),
        out_type=(
            jax.ShapeDtypeStruct((hw,), jnp.float32),
            jax.ShapeDtypeStruct((hsc, w), jnp.float32),
        ),
        scratch_types=[
            [pltpu.VMEM((chunk,), jnp.int32) for _ in range(2)],
            [pltpu.VMEM((chunk,), jnp.int32) for _ in range(2)],
            pltpu.VMEM((per_tile,), jnp.float32),
            [pltpu.VMEM((rpt, w), jnp.float32) for _ in range(4)],
            [pltpu.VMEM((rpt, w), jnp.float32) for _ in range(4)],
            pltpu.VMEM((rpt, w), jnp.float32),
            [pltpu.SemaphoreType.DMA for _ in range(4)],
            [pltpu.SemaphoreType.DMA for _ in range(2)],
        ],
    )
    def hist_sq(xs_hbm, ys_hbm, img_hbm, rew_hbm, cnt_hbm, sbot_hbm,
                xs_b, ys_b, hist_v, ibs, rbs, acc_v, sems, dsems):
        cid = lax.axis_index("c")
        sid = lax.axis_index("s")
        wid = sid * _NC + cid
        base = wid * per_tile
        row0 = row0_base + wid * rpt

        def fire(ch, k):
            pltpu.async_copy(img_hbm.at[ch, pl.ds(row0, rpt)], ibs[k], sems[k])
            pltpu.async_copy(rew_hbm.at[ch, pl.ds(row0, rpt)], rbs[k], sems[k])

        def drain(k):
            pltpu.make_async_copy(
                img_hbm.at[0, pl.ds(row0, rpt)], ibs[k], sems[k]
            ).wait()
            pltpu.make_async_copy(
                rew_hbm.at[0, pl.ds(row0, rpt)], rbs[k], sems[k]
            ).wait()

        for k in range(4):
            fire(k, k)

        zeros16 = jnp.zeros((_LANES,), jnp.float32)

        def zbody(k, carry):
            hist_v[pl.ds(k * _LANES, _LANES)] = zeros16
            return carry

        lax.fori_loop(0, per_tile // _LANES, zbody, 0)

        ones16 = jnp.ones((_LANES,), jnp.float32)
        n_chunks = n_pad // chunk

        def dfire(ci, k):
            pltpu.async_copy(xs_hbm.at[pl.ds(ci * chunk, chunk)], xs_b[k], dsems[k])
            pltpu.async_copy(ys_hbm.at[pl.ds(ci * chunk, chunk)], ys_b[k], dsems[k])

        def ddrain(k):
            pltpu.make_async_copy(xs_hbm.at[pl.ds(0, chunk)], xs_b[k], dsems[k]).wait()
            pltpu.make_async_copy(ys_hbm.at[pl.ds(0, chunk)], ys_b[k], dsems[k]).wait()

        dfire(0, 0)
        if n_chunks > 1:
            dfire(1, 1)
        for ci in range(n_chunks):
            k = ci % 2
            ddrain(k)
            xs_v, ys_v = xs_b[k], ys_b[k]

            @plsc.parallel_loop(0, chunk // _LANES, unroll=8)
            def _(i):
                sl = pl.ds(i * _LANES, _LANES)
                local = ys_v[sl] * w + xs_v[sl] - base
                mask = (local >= 0) & (local < per_tile)
                safe = jnp.minimum(jnp.maximum(local, 0), per_tile - 1)
                plsc.addupdate_scatter(hist_v, [safe], ones16, mask=mask)

            if ci + 2 < n_chunks:
                dfire(ci + 2, k)

        pltpu.sync_copy(hist_v, cnt_hbm.at[pl.ds(base, per_tile)])

        for r in range(rpt):

            @plsc.parallel_loop(0, w // _LANES, unroll=8)
            def _(g):
                acc_v[r, pl.ds(g * _LANES, _LANES)] = zeros16

        def accumulate(ib, rb):
            for r in range(rpt):

                @plsc.parallel_loop(0, w // _LANES, unroll=8)
                def _(g):
                    sl = pl.ds(g * _LANES, _LANES)
                    d = ib[r, sl] - rb[r, sl]
                    plsc.addupdate(acc_v.at[r, sl], d * d)

        def cbody(i, carry):
            for k in range(4):
                drain(k)
                accumulate(ibs[k], rbs[k])
                fire(4 * i + 4 + k, k)
            return carry

        lax.fori_loop(0, (c - 4) // 4, cbody, 0)
        for k in range(4):
            drain(k)
            accumulate(ibs[k], rbs[k])

        pltpu.sync_copy(acc_v, sbot_hbm.at[pl.ds(wid * rpt, rpt)])

    return hist_sq


def _sqdiff_body(img_ref, rew_ref, s_ref):
    d = img_ref[...] - rew_ref[...]
    s_ref[...] = jnp.sum(d * d, axis=0)


def _wsum_body(stop_ref, sbot_ref, cnt_ref, tot_ref):
    htop = stop_ref.shape[0]
    t = jnp.sum(stop_ref[...] * cnt_ref[:htop, :])
    b = jnp.sum(sbot_ref[...] * cnt_ref[htop:, :])
    tot_ref[0, 0] = t + b


def kernel(image, image_rewrite, dot_list_format):
    c, h, w = image.shape
    n = dot_list_format.shape[0]
    hw = h * w

    chunk = 10000
    if chunk % _LANES:
        chunk = ((chunk // _LANES) + 1) * _LANES
    n_pad = ((n + chunk - 1) // chunk) * chunk
    dots_t = dot_list_format.T
    if n_pad != n:
        fill = jnp.concatenate(
            [
                jnp.zeros((1, n_pad - n), jnp.int32),
                jnp.full((1, n_pad - n), h, jnp.int32),
            ],
            axis=0,
        )
        dots_t = jnp.concatenate([dots_t, fill], axis=1)

    hsc = _NW * max(1, (h // 4) // _NW)
    counts, s_bot = _make_hist_sqdiff(n_pad, chunk, h, w, c, hsc)(
        dots_t[0], dots_t[1], image, image_rewrite
    )
    counts2d = counts.reshape(h, w)

    htop = h - hsc
    br = 32
    s_top = pl.pallas_call(
        _sqdiff_body,
        grid=(htop // br,),
        in_specs=[
            pl.BlockSpec((c, br, w), lambda i: (0, i, 0)),
            pl.BlockSpec((c, br, w), lambda i: (0, i, 0)),
        ],
        out_specs=pl.BlockSpec((br, w), lambda i: (i, 0)),
        out_shape=jax.ShapeDtypeStruct((htop, w), jnp.float32),
    )(image, image_rewrite)

    tot = pl.pallas_call(
        _wsum_body,
        in_specs=[
            pl.BlockSpec((htop, w), lambda: (0, 0)),
            pl.BlockSpec((hsc, w), lambda: (0, 0)),
            pl.BlockSpec((h, w), lambda: (0, 0)),
        ],
        out_specs=pl.BlockSpec(memory_space=pltpu.SMEM),
        out_shape=jax.ShapeDtypeStruct((1, 1), jnp.float32),
    )(s_top, s_bot, counts2d)

    return tot[0, 0] / jnp.float32(n)
